# baseline (speedup 1.0000x reference)
import jax
import jax.numpy as jnp
from jax.experimental import pallas as pl
from jax.experimental.pallas import tpu as pltpu

_D_IN = 768
_D_H = 1024
_NL = 4
_NB = 8
_BLK = _D_H // _NB


def _body(q_ref, we_ref, be_ref, wp_ref, bp_ref, gp_ref, betap_ref,
          out_ref, x_ref, h_ref):
    i = pl.program_id(0)
    j = pl.program_id(1)
    off = pl.multiple_of(j * _BLK, _BLK)

    @pl.when(i == 0)
    def _enc_block():
        res = jax.lax.dot_general(
            q_ref[...], we_ref[...], (((1,), (1,)), ((), ())),
            preferred_element_type=jnp.float32)
        h_ref[:, pl.ds(off, _BLK)] = res

    @pl.when(jnp.logical_and(i == 0, j == _NB - 1))
    def _enc_finish():
        h = h_ref[...] + be_ref[...]
        n = jnp.sqrt(jnp.sum(h * h))
        x_ref[...] = h / jnp.maximum(n, 1e-12)

    @pl.when(i > 0)
    def _layer_block():
        res = jax.lax.dot_general(
            x_ref[...], wp_ref[0], (((1,), (1,)), ((), ())),
            preferred_element_type=jnp.float32) + bp_ref[0, 0]
        h_ref[:, pl.ds(off, _BLK)] = res

    @pl.when(jnp.logical_and(i > 0, j == _NB - 1))
    def _layer_finish():
        h = h_ref[...]
        h = 0.5 * h * (1.0 + jax.lax.erf(h * 0.7071067811865476))
        mu = jnp.mean(h, axis=-1, keepdims=True)
        var = jnp.mean((h - mu) * (h - mu), axis=-1, keepdims=True)
        h = (h - mu) / jnp.sqrt(var + 1e-5) * gp_ref[0] + betap_ref[0]
        x = x_ref[...] + h
        x_ref[...] = x

        @pl.when(i == _NL)
        def _finish():
            out_ref[...] = x


def kernel(query, context, W_enc, b_enc, Wp, bp, gp, betap):
    del context
    q2 = query.reshape(1, _D_IN)
    be2 = b_enc.reshape(1, _D_H)

    def we_idx(i, j):
        return (jnp.where(i == 0, j, _NB - 1), 0)

    def wp_idx(i, j):
        li = jnp.maximum(i - 1, 0)
        return (li, jnp.where(i == 0, 0, j), 0)

    def bp_idx(i, j):
        li = jnp.maximum(i - 1, 0)
        return (li, jnp.where(i == 0, 0, j), 0, 0)

    def ln_idx(i, j):
        return (jnp.maximum(i - 1, 0), 0, 0)

    out = pl.pallas_call(
        _body,
        grid=(_NL + 1, _NB),
        in_specs=[
            pl.BlockSpec((1, _D_IN), lambda i, j: (0, 0)),
            pl.BlockSpec((_BLK, _D_IN), we_idx),
            pl.BlockSpec((1, _D_H), lambda i, j: (0, 0)),
            pl.BlockSpec((1, _BLK, _D_H), wp_idx),
            pl.BlockSpec((1, 1, 1, _BLK), bp_idx),
            pl.BlockSpec((1, 1, _D_H), ln_idx),
            pl.BlockSpec((1, 1, _D_H), ln_idx),
        ],
        out_specs=pl.BlockSpec((1, _D_H), lambda i, j: (0, 0)),
        out_shape=jax.ShapeDtypeStruct((1, _D_H), jnp.float32),
        scratch_shapes=[
            pltpu.VMEM((1, _D_H), jnp.float32),
            pltpu.VMEM((1, _D_H), jnp.float32),
        ],
        compiler_params=pltpu.CompilerParams(
            dimension_semantics=("arbitrary", "arbitrary"),
        ),
    )(q2, W_enc, be2, Wp,
      bp.reshape(_NL, _NB, 1, _BLK),
      gp.reshape(_NL, 1, _D_H), betap.reshape(_NL, 1, _D_H))
    return out.reshape(_D_H)


# speedup vs baseline: 2.1011x; 2.1011x over previous
import jax
import jax.numpy as jnp
from jax.experimental import pallas as pl
from jax.experimental.pallas import tpu as pltpu

_D_IN = 768
_D_H = 1024
_NL = 4


def _body(q_ref, we_ref, be_ref, wp_ref, bp_ref, gp_ref, betap_ref,
          out_ref, x_ref):
    i = pl.program_id(0)

    @pl.when(i == 0)
    def _encode():
        q = q_ref[...]
        we = we_ref[...]
        h = jax.lax.dot_general(
            q, we, (((1,), (1,)), ((), ())),
            preferred_element_type=jnp.float32) + be_ref[...]
        n = jnp.sqrt(jnp.sum(h * h))
        x_ref[...] = h / jnp.maximum(n, 1e-12)

    x = x_ref[...]
    w = wp_ref[0]
    h = jax.lax.dot_general(
        x, w, (((1,), (1,)), ((), ())),
        preferred_element_type=jnp.float32) + bp_ref[0]
    h = 0.5 * h * (1.0 + jax.lax.erf(h * 0.7071067811865476))
    mu = jnp.mean(h, axis=-1, keepdims=True)
    var = jnp.mean((h - mu) * (h - mu), axis=-1, keepdims=True)
    h = (h - mu) / jnp.sqrt(var + 1e-5) * gp_ref[0] + betap_ref[0]
    x = x + h
    x_ref[...] = x

    @pl.when(i == _NL - 1)
    def _finish():
        out_ref[...] = x


def kernel(query, context, W_enc, b_enc, Wp, bp, gp, betap):
    del context
    q2 = query.reshape(1, _D_IN)
    be2 = b_enc.reshape(1, _D_H)
    out = pl.pallas_call(
        _body,
        grid=(_NL,),
        in_specs=[
            pl.BlockSpec((1, _D_IN), lambda i: (0, 0)),
            pl.BlockSpec((_D_H, _D_IN), lambda i: (0, 0)),
            pl.BlockSpec((1, _D_H), lambda i: (0, 0)),
            pl.BlockSpec((1, _D_H, _D_H), lambda i: (i, 0, 0)),
            pl.BlockSpec((1, 1, _D_H), lambda i: (i, 0, 0)),
            pl.BlockSpec((1, 1, _D_H), lambda i: (i, 0, 0)),
            pl.BlockSpec((1, 1, _D_H), lambda i: (i, 0, 0)),
        ],
        out_specs=pl.BlockSpec((1, _D_H), lambda i: (0, 0)),
        out_shape=jax.ShapeDtypeStruct((1, _D_H), jnp.float32),
        scratch_shapes=[pltpu.VMEM((1, _D_H), jnp.float32)],
        compiler_params=pltpu.CompilerParams(
            dimension_semantics=("arbitrary",),
        ),
    )(q2, W_enc, be2, Wp, bp.reshape(_NL, 1, _D_H), gp.reshape(_NL, 1, _D_H),
      betap.reshape(_NL, 1, _D_H))
    return out.reshape(_D_H)
